# baseline (speedup 1.0000x reference)
import jax
import jax.numpy as jnp
from jax.experimental import pallas as pl
from jax.experimental.pallas import tpu as pltpu

B = 128
DIM = 32768
HID = 64
STEP = 0.4
BAL = 1.0
TEMP = 1.0
EPS = 1e-10
TERM2 = 1.0 / (2.0 * STEP)

C = 4096
N = DIM // C

bf16 = jnp.bfloat16
f32 = jnp.float32


def _dot(a, b):
    return jax.lax.dot_general(a, b, (((1,), (0,)), ((), ())),
                               preferred_element_type=f32)


def _acc_chunks(acc_ref, vals):
    parts = [vals[:, k * 128:(k + 1) * 128] for k in range(C // 128)]
    while len(parts) > 1:
        parts = [parts[i] + parts[i + 1] for i in range(0, len(parts), 2)]
    acc_ref[...] += parts[0]


def _split_cat3(v):
    hi = v.astype(bf16)
    lo = (v - hi.astype(f32)).astype(bf16)
    return jnp.concatenate([hi, hi, lo], axis=1)


def _body(x_j, u_j, wcat_j, wtcat_j, b_j, u2,
          out_j,
          xdc,
          hv_ref, hdv_ref, hcat_ref, hdcat_ref,
          xbv, xdbv, lpfv, lprv, a_ref):
    p = pl.program_id(0)
    j = pl.program_id(1)
    cols = pl.ds(j * C, C)

    @pl.when(p == 0)
    def _phase0():
        @pl.when(j == 0)
        def _():
            hv_ref[...] = jnp.zeros_like(hv_ref)
            xbv[...] = jnp.zeros_like(xbv)

        x = x_j[...]
        hv_ref[...] += _dot(x.astype(bf16), wcat_j[...])
        _acc_chunks(xbv, x * b_j[...])

    @pl.when(p == 1)
    def _phase1():
        @pl.when(j == 0)
        def _():
            hdv_ref[...] = jnp.zeros_like(hdv_ref)
            xdbv[...] = jnp.zeros_like(xdbv)
            lpfv[...] = jnp.zeros_like(lpfv)
            hv = hv_ref[...]
            hcat_ref[...] = _split_cat3(hv[:, 0:HID] + hv[:, HID:2 * HID])

        grad = b_j[...] - _dot(hcat_ref[...], wtcat_j[...])
        x = x_j[...]
        z = BAL * (1.0 - 2.0 * x) * grad - TERM2
        w = jnp.exp(-z)
        ind = u_j[...] * (1.0 + w) < 1.0
        xd = jnp.where(ind, 1.0 - x, x)
        xdc[:, cols] = xd
        lw = jnp.log(1.0 + w)
        _acc_chunks(lpfv, jnp.where(ind, -lw, -(z + lw)))
        hdv_ref[...] += _dot(xd.astype(bf16), wcat_j[...])
        _acc_chunks(xdbv, xd * b_j[...])

    @pl.when(p == 2)
    def _phase2():
        @pl.when(j == 0)
        def _():
            lprv[...] = jnp.zeros_like(lprv)
            hdv = hdv_ref[...]
            hdcat_ref[...] = _split_cat3(hdv[:, 0:HID] + hdv[:, HID:2 * HID])

        grad_d = b_j[...] - _dot(hdcat_ref[...], wtcat_j[...])
        x = x_j[...]
        xd = xdc[:, cols]
        ind = jnp.abs(xd - x) > 0.5
        zr = BAL * (1.0 - 2.0 * xd) * grad_d - TERM2
        wr = jnp.exp(-zr)
        lwr = jnp.log(1.0 + wr)
        _acc_chunks(lprv, jnp.where(ind, -lwr, -(zr + lwr)))

        @pl.when(j == N - 1)
        def _():
            hv = hv_ref[...]
            h = hv[:, 0:HID] + hv[:, HID:2 * HID]
            hdv = hdv_ref[...]
            hd = hdv[:, 0:HID] + hdv[:, HID:2 * HID]
            xb = jnp.sum(xbv[...], axis=1, keepdims=True)
            xdb = jnp.sum(xdbv[...], axis=1, keepdims=True)
            lpf = jnp.sum(lpfv[...], axis=1, keepdims=True)
            lpr = jnp.sum(lprv[...], axis=1, keepdims=True)
            m = (xdb - 0.5 * jnp.sum(hd * hd, axis=1, keepdims=True)) \
                - (xb - 0.5 * jnp.sum(h * h, axis=1, keepdims=True))
            la = m * TEMP + lpr - lpf
            a_ref[...] = (jnp.log(u2[...] + EPS) < la).astype(f32)

    @pl.when(p == 3)
    def _phase3():
        out_j[...] = jnp.where(a_ref[...] > 0.5, xdc[:, cols], x_j[...])


def kernel(x, W, b, u, u2):
    W_hi = W.astype(bf16)
    W_lo = (W - W_hi.astype(f32)).astype(bf16)
    Wcat = jnp.concatenate([W_hi, W_lo], axis=1)
    Wtcat = jnp.concatenate([W_hi.T, W_lo.T, W_hi.T], axis=0)
    b2 = b.reshape(1, DIM)
    u2c = u2.reshape(B, 1)

    blk_x = pl.BlockSpec((B, C), lambda p, j: (0, j))
    blk_u = pl.BlockSpec((B, C), lambda p, j: (0, jnp.where(p == 1, j, 0)))
    blk_W = pl.BlockSpec((C, 2 * HID), lambda p, j: (jnp.where(p <= 1, j, 0), 0))
    blk_Wt = pl.BlockSpec((3 * HID, C),
                          lambda p, j: (0, jnp.where((p == 1) | (p == 2), j, 0)))
    blk_b = pl.BlockSpec((1, C), lambda p, j: (0, j))
    blk_u2 = pl.BlockSpec((B, 1), lambda p, j: (0, 0))
    blk_out = pl.BlockSpec((B, C), lambda p, j: (0, jnp.where(p == 3, j, 0)))

    return pl.pallas_call(
        _body,
        grid=(4, N),
        in_specs=[blk_x, blk_u, blk_W, blk_Wt, blk_b, blk_u2],
        out_specs=blk_out,
        out_shape=jax.ShapeDtypeStruct((B, DIM), f32),
        scratch_shapes=[
            pltpu.VMEM((B, DIM), f32),
            pltpu.VMEM((B, 2 * HID), f32),
            pltpu.VMEM((B, 2 * HID), f32),
            pltpu.VMEM((B, 3 * HID), bf16),
            pltpu.VMEM((B, 3 * HID), bf16),
            pltpu.VMEM((B, 128), f32),
            pltpu.VMEM((B, 128), f32),
            pltpu.VMEM((B, 128), f32),
            pltpu.VMEM((B, 128), f32),
            pltpu.VMEM((B, 1), f32),
        ],
    )(x, u, Wcat, Wtcat, b2, u2c)


# speedup vs baseline: 1.0576x; 1.0576x over previous
import jax
import jax.numpy as jnp
from jax.experimental import pallas as pl
from jax.experimental.pallas import tpu as pltpu

B = 128
DIM = 32768
HID = 64
STEP = 0.4
BAL = 1.0
TEMP = 1.0
EPS = 1e-10
TERM2 = 1.0 / (2.0 * STEP)

C = 4096
N = DIM // C

bf16 = jnp.bfloat16
f32 = jnp.float32


def _dot(a, b):
    return jax.lax.dot_general(a, b, (((1,), (0,)), ((), ())),
                               preferred_element_type=f32)


def _acc_chunks(acc_ref, vals):
    s = vals[:, 0:128]
    for k in range(1, C // 128):
        s = s + vals[:, k * 128:(k + 1) * 128]
    acc_ref[...] += s


def _split_cat3(v):
    hi = v.astype(bf16)
    lo = (v - hi.astype(f32)).astype(bf16)
    return jnp.concatenate([hi, hi, lo], axis=1)


def _body(x_j, u_j, wcat_j, wtcat_j, b_j, u2,
          out_j,
          xc, xdc,
          hv_ref, hdv_ref, hcat_ref, hdcat_ref,
          xbv, xdbv, lpfv, lprv, a_ref):
    p = pl.program_id(0)
    j = pl.program_id(1)
    cols = pl.ds(j * C, C)

    @pl.when(p == 0)
    def _phase0():
        @pl.when(j == 0)
        def _():
            hv_ref[...] = jnp.zeros_like(hv_ref)
            xbv[...] = jnp.zeros_like(xbv)

        x = x_j[...]
        x16 = x.astype(bf16)
        xc[:, cols] = x16
        hv_ref[...] += _dot(x16, wcat_j[...])
        _acc_chunks(xbv, x * b_j[...])

    @pl.when(p == 1)
    def _phase1():
        @pl.when(j == 0)
        def _():
            hdv_ref[...] = jnp.zeros_like(hdv_ref)
            xdbv[...] = jnp.zeros_like(xdbv)
            lpfv[...] = jnp.zeros_like(lpfv)
            hv = hv_ref[...]
            hcat_ref[...] = _split_cat3(hv[:, 0:HID] + hv[:, HID:2 * HID])

        grad = b_j[...] - _dot(hcat_ref[...], wtcat_j[...])
        x = xc[:, cols].astype(f32)
        z = BAL * (1.0 - 2.0 * x) * grad - TERM2
        w = jnp.exp(-z)
        ind = u_j[...] * (1.0 + w) < 1.0
        xd = jnp.where(ind, 1.0 - x, x)
        xdc[:, cols] = xd.astype(bf16)
        lw = jnp.log(1.0 + w)
        _acc_chunks(lpfv, jnp.where(ind, -lw, -(z + lw)))
        hdv_ref[...] += _dot(xd.astype(bf16), wcat_j[...])
        _acc_chunks(xdbv, xd * b_j[...])

    @pl.when(p == 2)
    def _phase2():
        @pl.when(j == 0)
        def _():
            lprv[...] = jnp.zeros_like(lprv)
            hdv = hdv_ref[...]
            hdcat_ref[...] = _split_cat3(hdv[:, 0:HID] + hdv[:, HID:2 * HID])

        grad_d = b_j[...] - _dot(hdcat_ref[...], wtcat_j[...])
        x = xc[:, cols].astype(f32)
        xd = xdc[:, cols].astype(f32)
        ind = jnp.abs(xd - x) > 0.5
        zr = BAL * (1.0 - 2.0 * xd) * grad_d - TERM2
        wr = jnp.exp(-zr)
        lwr = jnp.log(1.0 + wr)
        _acc_chunks(lprv, jnp.where(ind, -lwr, -(zr + lwr)))

        @pl.when(j == N - 1)
        def _():
            hv = hv_ref[...]
            h = hv[:, 0:HID] + hv[:, HID:2 * HID]
            hdv = hdv_ref[...]
            hd = hdv[:, 0:HID] + hdv[:, HID:2 * HID]
            xb = jnp.sum(xbv[...], axis=1, keepdims=True)
            xdb = jnp.sum(xdbv[...], axis=1, keepdims=True)
            lpf = jnp.sum(lpfv[...], axis=1, keepdims=True)
            lpr = jnp.sum(lprv[...], axis=1, keepdims=True)
            m = (xdb - 0.5 * jnp.sum(hd * hd, axis=1, keepdims=True)) \
                - (xb - 0.5 * jnp.sum(h * h, axis=1, keepdims=True))
            la = m * TEMP + lpr - lpf
            a_ref[...] = (jnp.log(u2[...] + EPS) < la).astype(f32)

    @pl.when(p == 3)
    def _phase3():
        x = xc[:, cols].astype(f32)
        xd = xdc[:, cols].astype(f32)
        out_j[...] = jnp.where(a_ref[...] > 0.5, xd, x)


def kernel(x, W, b, u, u2):
    W_hi = W.astype(bf16)
    W_lo = (W - W_hi.astype(f32)).astype(bf16)
    Wcat = jnp.concatenate([W_hi, W_lo], axis=1)
    Wtcat = jnp.concatenate([W_hi.T, W_lo.T, W_hi.T], axis=0)
    b2 = b.reshape(1, DIM)
    u2c = u2.reshape(B, 1)

    blk_x = pl.BlockSpec((B, C), lambda p, j: (0, jnp.where(p == 0, j, 0)))
    blk_u = pl.BlockSpec((B, C), lambda p, j: (0, jnp.where(p == 1, j, 0)))
    blk_W = pl.BlockSpec((C, 2 * HID), lambda p, j: (jnp.where(p <= 1, j, 0), 0))
    blk_Wt = pl.BlockSpec((3 * HID, C),
                          lambda p, j: (0, jnp.where((p == 1) | (p == 2), j, 0)))
    blk_b = pl.BlockSpec((1, C), lambda p, j: (0, jnp.where(p < 3, j, 0)))
    blk_u2 = pl.BlockSpec((B, 1), lambda p, j: (0, 0))
    blk_out = pl.BlockSpec((B, C), lambda p, j: (0, jnp.where(p == 3, j, 0)))

    return pl.pallas_call(
        _body,
        grid=(4, N),
        in_specs=[blk_x, blk_u, blk_W, blk_Wt, blk_b, blk_u2],
        out_specs=blk_out,
        out_shape=jax.ShapeDtypeStruct((B, DIM), f32),
        scratch_shapes=[
            pltpu.VMEM((B, DIM), bf16),
            pltpu.VMEM((B, DIM), bf16),
            pltpu.VMEM((B, 2 * HID), f32),
            pltpu.VMEM((B, 2 * HID), f32),
            pltpu.VMEM((B, 3 * HID), bf16),
            pltpu.VMEM((B, 3 * HID), bf16),
            pltpu.VMEM((B, 128), f32),
            pltpu.VMEM((B, 128), f32),
            pltpu.VMEM((B, 128), f32),
            pltpu.VMEM((B, 128), f32),
            pltpu.VMEM((B, 1), f32),
        ],
    )(x, u, Wcat, Wtcat, b2, u2c)
